# Initial kernel scaffold; baseline (speedup 1.0000x reference)
import jax
import jax.numpy as jnp
from jax.experimental import pallas as pl


def kernel(x, edge_index, W1, b1, W2, b2):
    raise NotImplementedError("write your pallas kernel here")


# speedup vs baseline: 9.9395x; 9.9395x over previous
import functools

import jax
import jax.numpy as jnp
from jax import lax
from jax.experimental import pallas as pl
from jax.experimental.pallas import tpu as pltpu
from jax.experimental.pallas import tpu_sc as plsc

N = 10000
E = 320000
D_IN = 128
D_HID = 256
D_OUT = 128

NSC = 2
NTILE = 16
NPAD = 10240
HIST_W = 128
K = 80


def _mesh():
    return plsc.VectorSubcoreMesh(core_axis_name="c", subcore_axis_name="s")


_EPW = E // (NSC * NTILE)
_HROWS = NPAD // NTILE


@functools.partial(
    pl.kernel,
    mesh=_mesh(),
    out_type=jax.ShapeDtypeStruct((NSC, NPAD, HIST_W), jnp.float32),
    scratch_types=[
        pltpu.VMEM((K,), jnp.int32),
        pltpu.VMEM((K, HIST_W), jnp.float32),
        pltpu.VMEM_SHARED((NPAD, HIST_W), jnp.float32),
    ],
)
def _sc_hist(idx, ones_hbm, zeros_hbm, out, didx, ones_v, hist):
    c = lax.axis_index("c")
    s = lax.axis_index("s")
    w = s * NSC + c
    hrow = pl.multiple_of(s * _HROWS, 8)
    pltpu.sync_copy(ones_hbm, ones_v)
    pltpu.sync_copy(zeros_hbm.at[pl.ds(hrow, _HROWS)],
                    hist.at[pl.ds(hrow, _HROWS)])
    plsc.subcore_barrier()
    base = 2 * E + w * _EPW

    def body(i, carry):
        off = pl.multiple_of(base + i * K, 8)
        pltpu.sync_copy(idx.at[pl.ds(off, K)], didx)
        pltpu.sync_copy(ones_v, hist.at[didx], add=True)
        return carry

    lax.fori_loop(0, _EPW // K, body, 0)
    plsc.subcore_barrier()
    pltpu.sync_copy(hist.at[pl.ds(hrow, _HROWS)],
                    out.at[c, pl.ds(hrow, _HROWS)])


def _make_sc_edge(dh):
    ept = E // NTILE
    nit = ept // K
    rpt = NPAD // NTILE

    @functools.partial(
        pl.kernel,
        mesh=_mesh(),
        out_type=jax.ShapeDtypeStruct((NSC, NPAD, dh), jnp.float32),
        scratch_types=[
            pltpu.VMEM((K,), jnp.int32),
            pltpu.VMEM((K,), jnp.int32),
            pltpu.VMEM((K, dh), jnp.float32),
            pltpu.VMEM_SHARED((NPAD, dh), jnp.float32),
            pltpu.SemaphoreType.DMA,
        ],
    )
    def edge_k(g, idx, out, sidx, didx, rows, acc, sem):
        c = lax.axis_index("c")
        s = lax.axis_index("s")
        arow = pl.multiple_of(s * rpt, 8)
        grow = pl.multiple_of(c * NPAD + s * rpt, 8)
        pltpu.sync_copy(g.at[pl.ds(grow, rpt)], acc.at[pl.ds(arow, rpt)])
        plsc.subcore_barrier()
        sbase = c * E + s * ept
        dbase = 2 * E + s * ept

        def body(i, carry):
            soff = pl.multiple_of(sbase + i * K, 8)
            doff = pl.multiple_of(dbase + i * K, 8)
            pltpu.sync_copy(idx.at[pl.ds(soff, K)], sidx)
            pltpu.sync_copy(idx.at[pl.ds(doff, K)], didx)
            pltpu.async_copy(g.at[sidx], rows, sem).wait()
            pltpu.sync_copy(rows, acc.at[didx], add=True)
            return carry

        lax.fori_loop(0, nit, body, 0)
        plsc.subcore_barrier()
        pltpu.sync_copy(acc.at[pl.ds(arow, rpt)],
                        out.at[c, pl.ds(arow, rpt)])

    return edge_k


_sc_edge_l1 = _make_sc_edge(D_HID // 2)


_EPT2 = E // (NSC * NTILE)


@functools.partial(
    pl.kernel,
    mesh=_mesh(),
    out_type=jax.ShapeDtypeStruct((NSC, NPAD, D_OUT), jnp.float32),
    scratch_types=[
        pltpu.VMEM((K,), jnp.int32),
        pltpu.VMEM((K,), jnp.int32),
        pltpu.VMEM((K, D_OUT), jnp.float32),
        pltpu.VMEM_SHARED((NPAD, D_OUT), jnp.float32),
        pltpu.SemaphoreType.DMA,
    ],
)
def _sc_edge_l2(g, idx, out, sidx, didx, rows, acc, sem):
    rpt = NPAD // NTILE
    c = lax.axis_index("c")
    s = lax.axis_index("s")
    arow = pl.multiple_of(s * rpt, 8)
    pltpu.sync_copy(g.at[pl.ds(arow, rpt)], acc.at[pl.ds(arow, rpt)])
    plsc.subcore_barrier()
    ebase = c * (E // 2) + s * _EPT2

    def body(i, carry):
        soff = pl.multiple_of(ebase + i * K, 8)
        doff = pl.multiple_of(2 * E + ebase + i * K, 8)
        pltpu.sync_copy(idx.at[pl.ds(soff, K)], sidx)
        pltpu.sync_copy(idx.at[pl.ds(doff, K)], didx)
        pltpu.async_copy(g.at[sidx], rows, sem).wait()
        pltpu.sync_copy(rows, acc.at[didx], add=True)
        return carry

    lax.fori_loop(0, _EPT2 // K, body, 0)
    plsc.subcore_barrier()
    pltpu.sync_copy(acc.at[pl.ds(arow, rpt)], out.at[c, pl.ds(arow, rpt)])


R = 2048
G = NPAD // R


def _tc_pre_body(x_ref, w1_ref, dinv_ref, out_ref):
    h = jnp.dot(x_ref[...], w1_ref[...],
                preferred_element_type=jnp.float32,
                precision=lax.Precision.HIGHEST)
    g = h * dinv_ref[...]
    hw = D_HID // 2
    out_ref[0] = g[:, :hw]
    out_ref[1] = g[:, hw:]


def _tc_mid_body(s1_ref, dinv_ref, b1_ref, w2_ref, out_ref):
    dinv = dinv_ref[...]
    hw = D_HID // 2
    x2a = jnp.maximum(s1_ref[0] * dinv + b1_ref[0, :hw], 0.0)
    x2b = jnp.maximum(s1_ref[1] * dinv + b1_ref[0, hw:], 0.0)
    h2 = (jnp.dot(x2a, w2_ref[:hw], preferred_element_type=jnp.float32,
                  precision=lax.Precision.HIGHEST)
          + jnp.dot(x2b, w2_ref[hw:], preferred_element_type=jnp.float32,
                    precision=lax.Precision.HIGHEST))
    out_ref[...] = h2 * dinv


def _tc_post_body(s2_ref, g2_ref, dinv_ref, b2_ref, out_ref):
    s2 = s2_ref[0] + s2_ref[1] - g2_ref[...]
    out_ref[...] = jnp.maximum(s2 * dinv_ref[...] + b2_ref[0], 0.0)


_tc_pre = pl.pallas_call(
    _tc_pre_body,
    grid=(G,),
    in_specs=[
        pl.BlockSpec((R, D_IN), lambda r: (r, 0)),
        pl.BlockSpec((D_IN, D_HID), lambda r: (0, 0)),
        pl.BlockSpec((R, 1), lambda r: (r, 0)),
    ],
    out_specs=pl.BlockSpec((NSC, R, D_HID // 2), lambda r: (0, r, 0)),
    out_shape=jax.ShapeDtypeStruct((NSC, NPAD, D_HID // 2), jnp.float32),
)
_tc_mid = pl.pallas_call(
    _tc_mid_body,
    grid=(G,),
    in_specs=[
        pl.BlockSpec((NSC, R, D_HID // 2), lambda r: (0, r, 0)),
        pl.BlockSpec((R, 1), lambda r: (r, 0)),
        pl.BlockSpec((1, D_HID), lambda r: (0, 0)),
        pl.BlockSpec((D_HID, D_OUT), lambda r: (0, 0)),
    ],
    out_specs=pl.BlockSpec((R, D_OUT), lambda r: (r, 0)),
    out_shape=jax.ShapeDtypeStruct((NPAD, D_OUT), jnp.float32),
)
_tc_post = pl.pallas_call(
    _tc_post_body,
    grid=(G,),
    in_specs=[
        pl.BlockSpec((NSC, R, D_OUT), lambda r: (0, r, 0)),
        pl.BlockSpec((R, D_OUT), lambda r: (r, 0)),
        pl.BlockSpec((R, 1), lambda r: (r, 0)),
        pl.BlockSpec((1, D_OUT), lambda r: (0, 0)),
    ],
    out_specs=pl.BlockSpec((R, D_OUT), lambda r: (r, 0)),
    out_shape=jax.ShapeDtypeStruct((NPAD, D_OUT), jnp.float32),
)


def kernel(x, edge_index, W1, b1, W2, b2):
    ei = edge_index.astype(jnp.int32)
    src = ei[0]
    dst = ei[1]
    idx = jnp.concatenate([src, src + NPAD, dst])
    ones_c = jnp.ones((K, HIST_W), jnp.float32)
    zeros_c = jnp.zeros((NPAD, HIST_W), jnp.float32)

    hist = _sc_hist(idx, ones_c, zeros_c)
    deg = 1.0 + hist[0, :, 0] + hist[1, :, 0]
    dinv = lax.rsqrt(deg)[:, None]

    x_pad = jnp.zeros((NPAD, D_IN), x.dtype).at[:N].set(x)
    g1 = _tc_pre(x_pad, W1, dinv)
    s1 = _sc_edge_l1(g1.reshape(2 * NPAD, D_HID // 2), idx)
    g2 = _tc_mid(s1, dinv, b1.reshape(1, -1), W2)
    s2 = _sc_edge_l2(g2, idx)
    return _tc_post(s2, g2, dinv, b2.reshape(1, -1))[:N]


# speedup vs baseline: 15.2612x; 1.5354x over previous
import functools

import jax
import jax.numpy as jnp
from jax import lax
from jax.experimental import pallas as pl
from jax.experimental.pallas import tpu as pltpu
from jax.experimental.pallas import tpu_sc as plsc

N = 10000
E = 320000
D_IN = 128
D_HID = 256
D_OUT = 128

NSC = 2
NTILE = 16
NPAD = 10240
HIST_W = 128
K = 80


def _mesh():
    return plsc.VectorSubcoreMesh(core_axis_name="c", subcore_axis_name="s")


_EPW = E // (NSC * NTILE)
_HROWS = NPAD // NTILE


@functools.partial(
    pl.kernel,
    mesh=_mesh(),
    out_type=jax.ShapeDtypeStruct((NSC, NPAD, HIST_W), jnp.float32),
    scratch_types=[
        pltpu.VMEM((2, K), jnp.int32),
        pltpu.VMEM((K, HIST_W), jnp.float32),
        pltpu.VMEM_SHARED((NPAD, HIST_W), jnp.float32),
        pltpu.SemaphoreType.DMA,
        pltpu.SemaphoreType.DMA,
    ],
)
def _sc_hist(idx, ones_hbm, zeros_hbm, out, didx, ones_v, hist, ss0, ss1):
    c = lax.axis_index("c")
    s = lax.axis_index("s")
    w = s * NSC + c
    hrow = pl.multiple_of(s * _HROWS, 8)
    pltpu.sync_copy(ones_hbm, ones_v)
    pltpu.sync_copy(zeros_hbm.at[pl.ds(hrow, _HROWS)],
                    hist.at[pl.ds(hrow, _HROWS)])
    plsc.subcore_barrier()
    base = 2 * E + w * _EPW
    nit = _EPW // K
    sems = (ss0, ss1)

    def load(chunk, b):
        off = pl.multiple_of(base + chunk * K, 8)
        pltpu.sync_copy(idx.at[pl.ds(off, K)], didx.at[b])

    def fire(b):
        pltpu.async_copy(ones_v, hist.at[didx.at[b]], sems[b], add=True)

    def drain(b):
        pltpu.make_async_copy(ones_v, hist.at[didx.at[b]], sems[b]).wait()

    load(0, 0)
    fire(0)

    def body(i, carry):
        @pl.when(i > 0)
        def _():
            drain(1)
        load(2 * i + 1, 1)
        fire(1)
        drain(0)
        load(2 * i + 2, 0)
        fire(0)
        return carry

    lax.fori_loop(0, (nit - 1) // 2, body, 0)
    drain(0)
    drain(1)
    plsc.subcore_barrier()
    pltpu.sync_copy(hist.at[pl.ds(hrow, _HROWS)],
                    out.at[c, pl.ds(hrow, _HROWS)])


def _make_sc_edge(dh, feature_split):
    ept = E // NTILE if feature_split else E // (NSC * NTILE)
    nit = ept // K
    rpt = NPAD // NTILE

    @functools.partial(
        pl.kernel,
        mesh=_mesh(),
        out_type=jax.ShapeDtypeStruct((NSC, NPAD, dh), jnp.float32),
        scratch_types=[
            pltpu.VMEM((2, K), jnp.int32),
            pltpu.VMEM((2, K), jnp.int32),
            pltpu.VMEM((2, K, dh), jnp.float32),
            pltpu.VMEM_SHARED((NPAD, dh), jnp.float32),
            pltpu.SemaphoreType.DMA,
            pltpu.SemaphoreType.DMA,
            pltpu.SemaphoreType.DMA,
            pltpu.SemaphoreType.DMA,
        ],
    )
    def edge_k(g, idx, out, sidx, didx, rows, acc, sg0, sg1, ss0, ss1):
        c = lax.axis_index("c")
        s = lax.axis_index("s")
        arow = pl.multiple_of(s * rpt, 8)
        if feature_split:
            grow = pl.multiple_of(c * NPAD + s * rpt, 8)
            sbase = c * E + s * ept
            dbase = 2 * E + s * ept
        else:
            grow = arow
            sbase = c * (E // 2) + s * ept
            dbase = 2 * E + c * (E // 2) + s * ept
        pltpu.sync_copy(g.at[pl.ds(grow, rpt)], acc.at[pl.ds(arow, rpt)])
        plsc.subcore_barrier()
        sgs = (sg0, sg1)
        sss = (ss0, ss1)

        def load_idx(chunk, b):
            soff = pl.multiple_of(sbase + chunk * K, 8)
            doff = pl.multiple_of(dbase + chunk * K, 8)
            pltpu.sync_copy(idx.at[pl.ds(soff, K)], sidx.at[b])
            pltpu.sync_copy(idx.at[pl.ds(doff, K)], didx.at[b])

        def fire_gather(b):
            pltpu.async_copy(g.at[sidx.at[b]], rows.at[b], sgs[b])

        def wait_gather(b):
            pltpu.make_async_copy(g.at[sidx.at[b]], rows.at[b], sgs[b]).wait()

        def fire_scatter(b):
            pltpu.async_copy(rows.at[b], acc.at[didx.at[b]], sss[b], add=True)

        def wait_scatter(b):
            pltpu.make_async_copy(rows.at[b], acc.at[didx.at[b]],
                                  sss[b]).wait()

        load_idx(0, 0)
        fire_gather(0)

        def body(i, carry):
            a = 2 * i
            wait_gather(0)
            fire_scatter(0)

            @pl.when(i > 0)
            def _():
                wait_scatter(1)

            load_idx(a + 1, 1)
            fire_gather(1)

            wait_scatter(0)

            @pl.when(a + 2 < nit)
            def _():
                load_idx(a + 2, 0)
                fire_gather(0)

            wait_gather(1)
            fire_scatter(1)
            return carry

        lax.fori_loop(0, nit // 2, body, 0)
        if nit % 2 == 1:
            wait_gather(0)
            fire_scatter(0)
            wait_scatter(0)
        wait_scatter(1)
        plsc.subcore_barrier()
        pltpu.sync_copy(acc.at[pl.ds(arow, rpt)],
                        out.at[c, pl.ds(arow, rpt)])

    return edge_k


_sc_edge_l1 = _make_sc_edge(D_HID // 2, feature_split=True)
_sc_edge_l2 = _make_sc_edge(D_OUT, feature_split=False)


R = 2048
G = NPAD // R


def _tc_pre_body(x_ref, w1_ref, dinv_ref, out_ref):
    h = jnp.dot(x_ref[...], w1_ref[...],
                preferred_element_type=jnp.float32,
                precision=lax.Precision.HIGHEST)
    g = h * dinv_ref[...]
    hw = D_HID // 2
    out_ref[0] = g[:, :hw]
    out_ref[1] = g[:, hw:]


def _tc_mid_body(s1_ref, dinv_ref, b1_ref, w2_ref, out_ref):
    dinv = dinv_ref[...]
    hw = D_HID // 2
    x2a = jnp.maximum(s1_ref[0] * dinv + b1_ref[0, :hw], 0.0)
    x2b = jnp.maximum(s1_ref[1] * dinv + b1_ref[0, hw:], 0.0)
    h2 = (jnp.dot(x2a, w2_ref[:hw], preferred_element_type=jnp.float32,
                  precision=lax.Precision.HIGHEST)
          + jnp.dot(x2b, w2_ref[hw:], preferred_element_type=jnp.float32,
                    precision=lax.Precision.HIGHEST))
    out_ref[...] = h2 * dinv


def _tc_post_body(s2_ref, g2_ref, dinv_ref, b2_ref, out_ref):
    s2 = s2_ref[0] + s2_ref[1] - g2_ref[...]
    out_ref[...] = jnp.maximum(s2 * dinv_ref[...] + b2_ref[0], 0.0)


_tc_pre = pl.pallas_call(
    _tc_pre_body,
    grid=(G,),
    in_specs=[
        pl.BlockSpec((R, D_IN), lambda r: (r, 0)),
        pl.BlockSpec((D_IN, D_HID), lambda r: (0, 0)),
        pl.BlockSpec((R, 1), lambda r: (r, 0)),
    ],
    out_specs=pl.BlockSpec((NSC, R, D_HID // 2), lambda r: (0, r, 0)),
    out_shape=jax.ShapeDtypeStruct((NSC, NPAD, D_HID // 2), jnp.float32),
)
_tc_mid = pl.pallas_call(
    _tc_mid_body,
    grid=(G,),
    in_specs=[
        pl.BlockSpec((NSC, R, D_HID // 2), lambda r: (0, r, 0)),
        pl.BlockSpec((R, 1), lambda r: (r, 0)),
        pl.BlockSpec((1, D_HID), lambda r: (0, 0)),
        pl.BlockSpec((D_HID, D_OUT), lambda r: (0, 0)),
    ],
    out_specs=pl.BlockSpec((R, D_OUT), lambda r: (r, 0)),
    out_shape=jax.ShapeDtypeStruct((NPAD, D_OUT), jnp.float32),
)
_tc_post = pl.pallas_call(
    _tc_post_body,
    grid=(G,),
    in_specs=[
        pl.BlockSpec((NSC, R, D_OUT), lambda r: (0, r, 0)),
        pl.BlockSpec((R, D_OUT), lambda r: (r, 0)),
        pl.BlockSpec((R, 1), lambda r: (r, 0)),
        pl.BlockSpec((1, D_OUT), lambda r: (0, 0)),
    ],
    out_specs=pl.BlockSpec((R, D_OUT), lambda r: (r, 0)),
    out_shape=jax.ShapeDtypeStruct((NPAD, D_OUT), jnp.float32),
)


def kernel(x, edge_index, W1, b1, W2, b2):
    ei = edge_index.astype(jnp.int32)
    src = ei[0]
    dst = ei[1]
    idx = jnp.concatenate([src, src + NPAD, dst])
    ones_c = jnp.ones((K, HIST_W), jnp.float32)
    zeros_c = jnp.zeros((NPAD, HIST_W), jnp.float32)

    hist = _sc_hist(idx, ones_c, zeros_c)
    deg = 1.0 + hist[0, :, 0] + hist[1, :, 0]
    dinv = lax.rsqrt(deg)[:, None]

    x_pad = jnp.zeros((NPAD, D_IN), x.dtype).at[:N].set(x)
    g1 = _tc_pre(x_pad, W1, dinv)
    s1 = _sc_edge_l1(g1.reshape(2 * NPAD, D_HID // 2), idx)
    g2 = _tc_mid(s1, dinv, b1.reshape(1, -1), W2)
    s2 = _sc_edge_l2(g2, idx)
    return _tc_post(s2, g2, dinv, b2.reshape(1, -1))[:N]


# speedup vs baseline: 17.4686x; 1.1446x over previous
import functools

import jax
import jax.numpy as jnp
from jax import lax
from jax.experimental import pallas as pl
from jax.experimental.pallas import tpu as pltpu
from jax.experimental.pallas import tpu_sc as plsc

N = 10000
E = 320000
D_IN = 128
D_HID = 256
D_OUT = 128

NSC = 2
NTILE = 16
NPAD = 10240
HIST_W = 128
K = 80


def _mesh():
    return plsc.VectorSubcoreMesh(core_axis_name="c", subcore_axis_name="s")


_EPW = E // (NSC * NTILE)
_HROWS = NPAD // NTILE


@functools.partial(
    pl.kernel,
    mesh=_mesh(),
    out_type=jax.ShapeDtypeStruct((NSC, NPAD, HIST_W), jnp.float32),
    scratch_types=[
        pltpu.VMEM((2, K), jnp.int32),
        pltpu.VMEM((K, HIST_W), jnp.float32),
        pltpu.VMEM_SHARED((NPAD, HIST_W), jnp.float32),
        pltpu.SemaphoreType.DMA,
        pltpu.SemaphoreType.DMA,
    ],
)
def _sc_hist(idx, ones_hbm, zeros_hbm, out, didx, ones_v, hist, ss0, ss1):
    c = lax.axis_index("c")
    s = lax.axis_index("s")
    w = s * NSC + c
    hrow = pl.multiple_of(s * _HROWS, 8)
    pltpu.sync_copy(ones_hbm, ones_v)
    pltpu.sync_copy(zeros_hbm.at[pl.ds(hrow, _HROWS)],
                    hist.at[pl.ds(hrow, _HROWS)])
    plsc.subcore_barrier()
    base = 2 * E + w * _EPW
    nit = _EPW // K
    sems = (ss0, ss1)

    def load(chunk, b):
        off = pl.multiple_of(base + chunk * K, 8)
        pltpu.sync_copy(idx.at[pl.ds(off, K)], didx.at[b])

    def fire(b):
        pltpu.async_copy(ones_v, hist.at[didx.at[b]], sems[b], add=True)

    def drain(b):
        pltpu.make_async_copy(ones_v, hist.at[didx.at[b]], sems[b]).wait()

    load(0, 0)
    fire(0)

    def body(i, carry):
        @pl.when(i > 0)
        def _():
            drain(1)
        load(2 * i + 1, 1)
        fire(1)
        drain(0)
        load(2 * i + 2, 0)
        fire(0)
        return carry

    lax.fori_loop(0, (nit - 1) // 2, body, 0)
    drain(0)
    drain(1)
    plsc.subcore_barrier()
    pltpu.sync_copy(hist.at[pl.ds(hrow, _HROWS)],
                    out.at[c, pl.ds(hrow, _HROWS)])


def _make_sc_edge(dh, feature_split):
    ept = E // NTILE if feature_split else E // (NSC * NTILE)
    nit = ept // K
    rpt = NPAD // NTILE
    NB = 4

    @functools.partial(
        pl.kernel,
        mesh=_mesh(),
        out_type=jax.ShapeDtypeStruct((NSC, NPAD, dh), jnp.float32),
        scratch_types=[
            pltpu.VMEM((NB, K), jnp.int32),
            pltpu.VMEM((NB, K), jnp.int32),
            pltpu.VMEM((NB, K, dh), jnp.float32),
            pltpu.VMEM_SHARED((NPAD, dh), jnp.float32),
        ] + [pltpu.SemaphoreType.DMA] * (2 * NB),
    )
    def edge_k(g, idx, out, sidx, didx, rows, acc, *sems):
        c = lax.axis_index("c")
        s = lax.axis_index("s")
        arow = pl.multiple_of(s * rpt, 8)
        if feature_split:
            grow = pl.multiple_of(c * NPAD + s * rpt, 8)
            sbase = c * E + s * ept
            dbase = 2 * E + s * ept
        else:
            grow = arow
            sbase = c * (E // 2) + s * ept
            dbase = 2 * E + c * (E // 2) + s * ept
        pltpu.sync_copy(g.at[pl.ds(grow, rpt)], acc.at[pl.ds(arow, rpt)])
        plsc.subcore_barrier()
        sgs = sems[:NB]
        sss = sems[NB:]

        def load_idx(chunk, b):
            soff = pl.multiple_of(sbase + chunk * K, 8)
            doff = pl.multiple_of(dbase + chunk * K, 8)
            pltpu.sync_copy(idx.at[pl.ds(soff, K)], sidx.at[b])
            pltpu.sync_copy(idx.at[pl.ds(doff, K)], didx.at[b])

        def fire_gather(b):
            pltpu.async_copy(g.at[sidx.at[b]], rows.at[b], sgs[b])

        def wait_gather(b):
            pltpu.make_async_copy(g.at[sidx.at[b]], rows.at[b], sgs[b]).wait()

        def fire_scatter(b):
            pltpu.async_copy(rows.at[b], acc.at[didx.at[b]], sss[b], add=True)

        def wait_scatter(b):
            pltpu.make_async_copy(rows.at[b], acc.at[didx.at[b]],
                                  sss[b]).wait()

        rings = (nit + NB - 1) // NB

        for b in range(NB):
            load_idx(b, b)
            fire_gather(b)

        def body(step, carry):
            base_c = step * NB
            for b in range(NB):
                chunk = base_c + b

                @pl.when(chunk < nit)
                def _(b=b):
                    wait_gather(b)
                    fire_scatter(b)
            for b in range(NB):
                nxt = base_c + NB + b

                @pl.when(nxt < nit)
                def _(b=b, nxt=nxt):
                    wait_scatter(b)
                    load_idx(nxt, b)
                    fire_gather(b)
            return carry

        lax.fori_loop(0, rings, body, 0)
        for b in range(NB):
            wait_scatter(b)
        plsc.subcore_barrier()
        pltpu.sync_copy(acc.at[pl.ds(arow, rpt)],
                        out.at[c, pl.ds(arow, rpt)])

    return edge_k


_sc_edge_l1 = _make_sc_edge(D_HID // 2, feature_split=True)
_sc_edge_l2 = _make_sc_edge(D_OUT, feature_split=False)


R = 2048
G = NPAD // R


def _tc_pre_body(x_ref, w1_ref, dinv_ref, out_ref):
    h = jnp.dot(x_ref[...], w1_ref[...],
                preferred_element_type=jnp.float32,
                precision=lax.Precision.HIGHEST)
    g = h * dinv_ref[...]
    hw = D_HID // 2
    out_ref[0] = g[:, :hw]
    out_ref[1] = g[:, hw:]


def _tc_mid_body(s1_ref, dinv_ref, b1_ref, w2_ref, out_ref):
    dinv = dinv_ref[...]
    hw = D_HID // 2
    x2a = jnp.maximum(s1_ref[0] * dinv + b1_ref[0, :hw], 0.0)
    x2b = jnp.maximum(s1_ref[1] * dinv + b1_ref[0, hw:], 0.0)
    h2 = (jnp.dot(x2a, w2_ref[:hw], preferred_element_type=jnp.float32,
                  precision=lax.Precision.HIGHEST)
          + jnp.dot(x2b, w2_ref[hw:], preferred_element_type=jnp.float32,
                    precision=lax.Precision.HIGHEST))
    out_ref[...] = h2 * dinv


def _tc_post_body(s2_ref, g2_ref, dinv_ref, b2_ref, out_ref):
    s2 = s2_ref[0] + s2_ref[1] - g2_ref[...]
    out_ref[...] = jnp.maximum(s2 * dinv_ref[...] + b2_ref[0], 0.0)


_tc_pre = pl.pallas_call(
    _tc_pre_body,
    grid=(G,),
    in_specs=[
        pl.BlockSpec((R, D_IN), lambda r: (r, 0)),
        pl.BlockSpec((D_IN, D_HID), lambda r: (0, 0)),
        pl.BlockSpec((R, 1), lambda r: (r, 0)),
    ],
    out_specs=pl.BlockSpec((NSC, R, D_HID // 2), lambda r: (0, r, 0)),
    out_shape=jax.ShapeDtypeStruct((NSC, NPAD, D_HID // 2), jnp.float32),
)
_tc_mid = pl.pallas_call(
    _tc_mid_body,
    grid=(G,),
    in_specs=[
        pl.BlockSpec((NSC, R, D_HID // 2), lambda r: (0, r, 0)),
        pl.BlockSpec((R, 1), lambda r: (r, 0)),
        pl.BlockSpec((1, D_HID), lambda r: (0, 0)),
        pl.BlockSpec((D_HID, D_OUT), lambda r: (0, 0)),
    ],
    out_specs=pl.BlockSpec((R, D_OUT), lambda r: (r, 0)),
    out_shape=jax.ShapeDtypeStruct((NPAD, D_OUT), jnp.float32),
)
_tc_post = pl.pallas_call(
    _tc_post_body,
    grid=(G,),
    in_specs=[
        pl.BlockSpec((NSC, R, D_OUT), lambda r: (0, r, 0)),
        pl.BlockSpec((R, D_OUT), lambda r: (r, 0)),
        pl.BlockSpec((R, 1), lambda r: (r, 0)),
        pl.BlockSpec((1, D_OUT), lambda r: (0, 0)),
    ],
    out_specs=pl.BlockSpec((R, D_OUT), lambda r: (r, 0)),
    out_shape=jax.ShapeDtypeStruct((NPAD, D_OUT), jnp.float32),
)


def kernel(x, edge_index, W1, b1, W2, b2):
    ei = edge_index.astype(jnp.int32)
    src = ei[0]
    dst = ei[1]
    idx = jnp.concatenate([src, src + NPAD, dst])
    ones_c = jnp.ones((K, HIST_W), jnp.float32)
    zeros_c = jnp.zeros((NPAD, HIST_W), jnp.float32)

    hist = _sc_hist(idx, ones_c, zeros_c)
    deg = 1.0 + hist[0, :, 0] + hist[1, :, 0]
    dinv = lax.rsqrt(deg)[:, None]

    x_pad = jnp.zeros((NPAD, D_IN), x.dtype).at[:N].set(x)
    g1 = _tc_pre(x_pad, W1, dinv)
    s1 = _sc_edge_l1(g1.reshape(2 * NPAD, D_HID // 2), idx)
    g2 = _tc_mid(s1, dinv, b1.reshape(1, -1), W2)
    s2 = _sc_edge_l2(g2, idx)
    return _tc_post(s2, g2, dinv, b2.reshape(1, -1))[:N]


# speedup vs baseline: 20.2233x; 1.1577x over previous
import functools

import jax
import jax.numpy as jnp
from jax import lax
from jax.experimental import pallas as pl
from jax.experimental.pallas import tpu as pltpu
from jax.experimental.pallas import tpu_sc as plsc

N = 10000
E = 320000
D_IN = 128
D_HID = 256
D_OUT = 128

NSC = 2
NTILE = 16
NPAD = 10240
HIST_W = 128
K = 80


def _mesh():
    return plsc.VectorSubcoreMesh(core_axis_name="c", subcore_axis_name="s")


_EPW = E // (NSC * NTILE)
_HROWS = NPAD // NTILE


@functools.partial(
    pl.kernel,
    mesh=_mesh(),
    out_type=jax.ShapeDtypeStruct((NSC, NPAD, HIST_W), jnp.float32),
    scratch_types=[
        pltpu.VMEM((2, K), jnp.int32),
        pltpu.VMEM((K, HIST_W), jnp.float32),
        pltpu.VMEM_SHARED((NPAD, HIST_W), jnp.float32),
        pltpu.SemaphoreType.DMA,
        pltpu.SemaphoreType.DMA,
    ],
)
def _sc_hist(idx, ones_hbm, zeros_hbm, out, didx, ones_v, hist, ss0, ss1):
    c = lax.axis_index("c")
    s = lax.axis_index("s")
    w = s * NSC + c
    hrow = pl.multiple_of(s * _HROWS, 8)
    pltpu.sync_copy(ones_hbm, ones_v)
    pltpu.sync_copy(zeros_hbm.at[pl.ds(hrow, _HROWS)],
                    hist.at[pl.ds(hrow, _HROWS)])
    plsc.subcore_barrier()
    base = 2 * E + w * _EPW
    nit = _EPW // K
    sems = (ss0, ss1)

    def load(chunk, b):
        off = pl.multiple_of(base + chunk * K, 8)
        pltpu.sync_copy(idx.at[pl.ds(off, K)], didx.at[b])

    def fire(b):
        pltpu.async_copy(ones_v, hist.at[didx.at[b]], sems[b], add=True)

    def drain(b):
        pltpu.make_async_copy(ones_v, hist.at[didx.at[b]], sems[b]).wait()

    load(0, 0)
    fire(0)

    def body(i, carry):
        @pl.when(i > 0)
        def _():
            drain(1)
        load(2 * i + 1, 1)
        fire(1)
        drain(0)
        load(2 * i + 2, 0)
        fire(0)
        return carry

    lax.fori_loop(0, (nit - 1) // 2, body, 0)
    drain(0)
    drain(1)
    plsc.subcore_barrier()
    pltpu.sync_copy(hist.at[pl.ds(hrow, _HROWS)],
                    out.at[c, pl.ds(hrow, _HROWS)])


def _make_sc_edge(dh, feature_split):
    ept = E // NTILE if feature_split else E // (NSC * NTILE)
    nit = ept // K
    rpt = NPAD // NTILE
    NB = 4

    @functools.partial(
        pl.kernel,
        mesh=_mesh(),
        out_type=jax.ShapeDtypeStruct((NSC, NPAD, dh), jnp.float32),
        scratch_types=[
            pltpu.VMEM((NB, K), jnp.int32),
            pltpu.VMEM((NB, K), jnp.int32),
            pltpu.VMEM((NB, K, dh), jnp.float32),
            pltpu.VMEM_SHARED((NPAD, dh), jnp.float32),
        ] + [pltpu.SemaphoreType.DMA] * (3 * NB),
    )
    def edge_k(g, idx, out, sidx, didx, rows, acc, *sems):
        c = lax.axis_index("c")
        s = lax.axis_index("s")
        arow = pl.multiple_of(s * rpt, 8)
        if feature_split:
            grow = pl.multiple_of(c * NPAD + s * rpt, 8)
            sbase = c * E + s * ept
            dbase = 2 * E + s * ept
        else:
            grow = arow
            sbase = c * (E // 2) + s * ept
            dbase = 2 * E + c * (E // 2) + s * ept
        pltpu.sync_copy(g.at[pl.ds(grow, rpt)], acc.at[pl.ds(arow, rpt)])
        plsc.subcore_barrier()
        sgs = sems[:NB]
        sss = sems[NB:2 * NB]
        sis = sems[2 * NB:]

        def fire_idx(chunk, b):
            soff = pl.multiple_of(sbase + chunk * K, 8)
            doff = pl.multiple_of(dbase + chunk * K, 8)
            pltpu.async_copy(idx.at[pl.ds(soff, K)], sidx.at[b], sis[b])
            pltpu.async_copy(idx.at[pl.ds(doff, K)], didx.at[b], sis[b])

        def wait_idx(chunk, b):
            soff = pl.multiple_of(sbase + chunk * K, 8)
            doff = pl.multiple_of(dbase + chunk * K, 8)
            pltpu.make_async_copy(idx.at[pl.ds(soff, K)], sidx.at[b],
                                  sis[b]).wait()
            pltpu.make_async_copy(idx.at[pl.ds(doff, K)], didx.at[b],
                                  sis[b]).wait()

        def fire_gather(b):
            pltpu.async_copy(g.at[sidx.at[b]], rows.at[b], sgs[b])

        def wait_gather(b):
            pltpu.make_async_copy(g.at[sidx.at[b]], rows.at[b], sgs[b]).wait()

        def fire_scatter(b):
            pltpu.async_copy(rows.at[b], acc.at[didx.at[b]], sss[b], add=True)

        def wait_scatter(b):
            pltpu.make_async_copy(rows.at[b], acc.at[didx.at[b]],
                                  sss[b]).wait()

        rings = (nit + NB - 1) // NB

        for b in range(NB):
            fire_idx(b, b)
        for b in range(NB):
            wait_idx(b, b)
            fire_gather(b)

        def body(step, carry):
            base_c = step * NB
            for b in range(NB):
                chunk = base_c + b

                @pl.when(chunk < nit)
                def _(b=b):
                    wait_gather(b)
                    fire_scatter(b)
            for b in range(NB):
                nxt = base_c + NB + b

                @pl.when(nxt < nit)
                def _(b=b, nxt=nxt):
                    wait_scatter(b)
                    fire_idx(nxt, b)
            for b in range(NB):
                nxt = base_c + NB + b

                @pl.when(nxt < nit)
                def _(b=b, nxt=nxt):
                    wait_idx(nxt, b)
                    fire_gather(b)
            return carry

        lax.fori_loop(0, rings, body, 0)
        for b in range(NB):
            wait_scatter(b)
        plsc.subcore_barrier()
        pltpu.sync_copy(acc.at[pl.ds(arow, rpt)],
                        out.at[c, pl.ds(arow, rpt)])

    return edge_k


_sc_edge_l1 = _make_sc_edge(D_HID // 2, feature_split=True)
_sc_edge_l2 = _make_sc_edge(D_OUT, feature_split=False)


R = 2048
G = NPAD // R


def _tc_pre_body(x_ref, w1_ref, dinv_ref, out_ref):
    h = jnp.dot(x_ref[...], w1_ref[...],
                preferred_element_type=jnp.float32,
                precision=lax.Precision.HIGHEST)
    g = h * dinv_ref[...]
    hw = D_HID // 2
    out_ref[0] = g[:, :hw]
    out_ref[1] = g[:, hw:]


def _tc_mid_body(s1_ref, dinv_ref, b1_ref, w2_ref, out_ref):
    dinv = dinv_ref[...]
    hw = D_HID // 2
    x2a = jnp.maximum(s1_ref[0] * dinv + b1_ref[0, :hw], 0.0)
    x2b = jnp.maximum(s1_ref[1] * dinv + b1_ref[0, hw:], 0.0)
    h2 = (jnp.dot(x2a, w2_ref[:hw], preferred_element_type=jnp.float32,
                  precision=lax.Precision.HIGHEST)
          + jnp.dot(x2b, w2_ref[hw:], preferred_element_type=jnp.float32,
                    precision=lax.Precision.HIGHEST))
    out_ref[...] = h2 * dinv


def _tc_post_body(s2_ref, g2_ref, dinv_ref, b2_ref, out_ref):
    s2 = s2_ref[0] + s2_ref[1] - g2_ref[...]
    out_ref[...] = jnp.maximum(s2 * dinv_ref[...] + b2_ref[0], 0.0)


_tc_pre = pl.pallas_call(
    _tc_pre_body,
    grid=(G,),
    in_specs=[
        pl.BlockSpec((R, D_IN), lambda r: (r, 0)),
        pl.BlockSpec((D_IN, D_HID), lambda r: (0, 0)),
        pl.BlockSpec((R, 1), lambda r: (r, 0)),
    ],
    out_specs=pl.BlockSpec((NSC, R, D_HID // 2), lambda r: (0, r, 0)),
    out_shape=jax.ShapeDtypeStruct((NSC, NPAD, D_HID // 2), jnp.float32),
)
_tc_mid = pl.pallas_call(
    _tc_mid_body,
    grid=(G,),
    in_specs=[
        pl.BlockSpec((NSC, R, D_HID // 2), lambda r: (0, r, 0)),
        pl.BlockSpec((R, 1), lambda r: (r, 0)),
        pl.BlockSpec((1, D_HID), lambda r: (0, 0)),
        pl.BlockSpec((D_HID, D_OUT), lambda r: (0, 0)),
    ],
    out_specs=pl.BlockSpec((R, D_OUT), lambda r: (r, 0)),
    out_shape=jax.ShapeDtypeStruct((NPAD, D_OUT), jnp.float32),
)
_tc_post = pl.pallas_call(
    _tc_post_body,
    grid=(G,),
    in_specs=[
        pl.BlockSpec((NSC, R, D_OUT), lambda r: (0, r, 0)),
        pl.BlockSpec((R, D_OUT), lambda r: (r, 0)),
        pl.BlockSpec((R, 1), lambda r: (r, 0)),
        pl.BlockSpec((1, D_OUT), lambda r: (0, 0)),
    ],
    out_specs=pl.BlockSpec((R, D_OUT), lambda r: (r, 0)),
    out_shape=jax.ShapeDtypeStruct((NPAD, D_OUT), jnp.float32),
)


def kernel(x, edge_index, W1, b1, W2, b2):
    ei = edge_index.astype(jnp.int32)
    src = ei[0]
    dst = ei[1]
    idx = jnp.concatenate([src, src + NPAD, dst])
    ones_c = jnp.ones((K, HIST_W), jnp.float32)
    zeros_c = jnp.zeros((NPAD, HIST_W), jnp.float32)

    hist = _sc_hist(idx, ones_c, zeros_c)
    deg = 1.0 + hist[0, :, 0] + hist[1, :, 0]
    dinv = lax.rsqrt(deg)[:, None]

    x_pad = jnp.zeros((NPAD, D_IN), x.dtype).at[:N].set(x)
    g1 = _tc_pre(x_pad, W1, dinv)
    s1 = _sc_edge_l1(g1.reshape(2 * NPAD, D_HID // 2), idx)
    g2 = _tc_mid(s1, dinv, b1.reshape(1, -1), W2)
    s2 = _sc_edge_l2(g2, idx)
    return _tc_post(s2, g2, dinv, b2.reshape(1, -1))[:N]


# speedup vs baseline: 20.3232x; 1.0049x over previous
import functools

import jax
import jax.numpy as jnp
from jax import lax
from jax.experimental import pallas as pl
from jax.experimental.pallas import tpu as pltpu
from jax.experimental.pallas import tpu_sc as plsc

N = 10000
E = 320000
D_IN = 128
D_HID = 256
D_OUT = 128

NSC = 2
NTILE = 16
NPAD = 10240
HIST_W = 128
K = 80


def _mesh():
    return plsc.VectorSubcoreMesh(core_axis_name="c", subcore_axis_name="s")


_EPW = E // (NSC * NTILE)
_HROWS = NPAD // NTILE


@functools.partial(
    pl.kernel,
    mesh=_mesh(),
    out_type=jax.ShapeDtypeStruct((NSC, NPAD, HIST_W), jnp.float32),
    scratch_types=[
        pltpu.VMEM((4, K), jnp.int32),
        pltpu.VMEM((K, HIST_W), jnp.float32),
        pltpu.VMEM_SHARED((NPAD, HIST_W), jnp.float32),
    ] + [pltpu.SemaphoreType.DMA] * 8,
)
def _sc_hist(idx, ones_hbm, zeros_hbm, out, didx, ones_v, hist, *sems):
    NB = 4
    c = lax.axis_index("c")
    s = lax.axis_index("s")
    w = s * NSC + c
    hrow = pl.multiple_of(s * _HROWS, 8)
    pltpu.sync_copy(ones_hbm, ones_v)
    pltpu.sync_copy(zeros_hbm.at[pl.ds(hrow, _HROWS)],
                    hist.at[pl.ds(hrow, _HROWS)])
    plsc.subcore_barrier()
    base = 2 * E + w * _EPW
    nit = _EPW // K
    sss = sems[:NB]
    sis = sems[NB:]

    def fire_idx(chunk, b):
        off = pl.multiple_of(base + chunk * K, 8)
        pltpu.async_copy(idx.at[pl.ds(off, K)], didx.at[b], sis[b])

    def wait_idx(chunk, b):
        off = pl.multiple_of(base + chunk * K, 8)
        pltpu.make_async_copy(idx.at[pl.ds(off, K)], didx.at[b],
                              sis[b]).wait()

    def fire_scatter(b):
        pltpu.async_copy(ones_v, hist.at[didx.at[b]], sss[b], add=True)

    def wait_scatter(b):
        pltpu.make_async_copy(ones_v, hist.at[didx.at[b]], sss[b]).wait()

    for b in range(NB):
        fire_idx(b, b)

    def body(step, carry):
        base_c = step * NB
        for b in range(NB):
            chunk = base_c + b

            @pl.when(chunk < nit)
            def _(b=b, chunk=chunk):
                wait_idx(chunk, b)
                fire_scatter(b)
        for b in range(NB):
            nxt = base_c + NB + b

            @pl.when(nxt < nit)
            def _(b=b, nxt=nxt):
                wait_scatter(b)
                fire_idx(nxt, b)
        return carry

    lax.fori_loop(0, (nit + NB - 1) // NB, body, 0)
    for b in range(NB):
        wait_scatter(b)
    plsc.subcore_barrier()
    pltpu.sync_copy(hist.at[pl.ds(hrow, _HROWS)],
                    out.at[c, pl.ds(hrow, _HROWS)])


def _make_sc_edge(dh, feature_split):
    ept = E // NTILE if feature_split else E // (NSC * NTILE)
    nit = ept // K
    rpt = NPAD // NTILE
    NB = 4

    @functools.partial(
        pl.kernel,
        mesh=_mesh(),
        out_type=jax.ShapeDtypeStruct((NSC, NPAD, dh), jnp.float32),
        scratch_types=[
            pltpu.VMEM((NB, K), jnp.int32),
            pltpu.VMEM((NB, K), jnp.int32),
            pltpu.VMEM((NB, K, dh), jnp.float32),
            pltpu.VMEM_SHARED((NPAD, dh), jnp.float32),
        ] + [pltpu.SemaphoreType.DMA] * (3 * NB),
    )
    def edge_k(g, idx, out, sidx, didx, rows, acc, *sems):
        c = lax.axis_index("c")
        s = lax.axis_index("s")
        arow = pl.multiple_of(s * rpt, 8)
        if feature_split:
            grow = pl.multiple_of(c * NPAD + s * rpt, 8)
            sbase = c * E + s * ept
            dbase = 2 * E + s * ept
        else:
            grow = arow
            sbase = c * (E // 2) + s * ept
            dbase = 2 * E + c * (E // 2) + s * ept
        pltpu.sync_copy(g.at[pl.ds(grow, rpt)], acc.at[pl.ds(arow, rpt)])
        plsc.subcore_barrier()
        sgs = sems[:NB]
        sss = sems[NB:2 * NB]
        sis = sems[2 * NB:]

        def fire_idx(chunk, b):
            soff = pl.multiple_of(sbase + chunk * K, 8)
            doff = pl.multiple_of(dbase + chunk * K, 8)
            pltpu.async_copy(idx.at[pl.ds(soff, K)], sidx.at[b], sis[b])
            pltpu.async_copy(idx.at[pl.ds(doff, K)], didx.at[b], sis[b])

        def wait_idx(chunk, b):
            soff = pl.multiple_of(sbase + chunk * K, 8)
            doff = pl.multiple_of(dbase + chunk * K, 8)
            pltpu.make_async_copy(idx.at[pl.ds(soff, K)], sidx.at[b],
                                  sis[b]).wait()
            pltpu.make_async_copy(idx.at[pl.ds(doff, K)], didx.at[b],
                                  sis[b]).wait()

        def fire_gather(b):
            pltpu.async_copy(g.at[sidx.at[b]], rows.at[b], sgs[b])

        def wait_gather(b):
            pltpu.make_async_copy(g.at[sidx.at[b]], rows.at[b], sgs[b]).wait()

        def fire_scatter(b):
            pltpu.async_copy(rows.at[b], acc.at[didx.at[b]], sss[b], add=True)

        def wait_scatter(b):
            pltpu.make_async_copy(rows.at[b], acc.at[didx.at[b]],
                                  sss[b]).wait()

        rings = (nit + NB - 1) // NB

        for b in range(NB):
            fire_idx(b, b)
        for b in range(NB):
            wait_idx(b, b)
            fire_gather(b)

        def body(step, carry):
            base_c = step * NB
            for b in range(NB):
                chunk = base_c + b

                @pl.when(chunk < nit)
                def _(b=b):
                    wait_gather(b)
                    fire_scatter(b)
            for b in range(NB):
                nxt = base_c + NB + b

                @pl.when(nxt < nit)
                def _(b=b, nxt=nxt):
                    wait_scatter(b)
                    fire_idx(nxt, b)
            for b in range(NB):
                nxt = base_c + NB + b

                @pl.when(nxt < nit)
                def _(b=b, nxt=nxt):
                    wait_idx(nxt, b)
                    fire_gather(b)
            return carry

        lax.fori_loop(0, rings, body, 0)
        for b in range(NB):
            wait_scatter(b)
        plsc.subcore_barrier()
        pltpu.sync_copy(acc.at[pl.ds(arow, rpt)],
                        out.at[c, pl.ds(arow, rpt)])

    return edge_k


_sc_edge_l1 = _make_sc_edge(D_HID // 2, feature_split=True)
_sc_edge_l2 = _make_sc_edge(D_OUT, feature_split=False)


R = 2048
G = NPAD // R


def _tc_pre_body(x_ref, w1_ref, dinv_ref, out_ref):
    h = jnp.dot(x_ref[...], w1_ref[...],
                preferred_element_type=jnp.float32,
                precision=lax.Precision.HIGHEST)
    g = h * dinv_ref[...]
    hw = D_HID // 2
    out_ref[0] = g[:, :hw]
    out_ref[1] = g[:, hw:]


def _tc_mid_body(s1_ref, dinv_ref, b1_ref, w2_ref, out_ref):
    dinv = dinv_ref[...]
    hw = D_HID // 2
    x2a = jnp.maximum(s1_ref[0] * dinv + b1_ref[0, :hw], 0.0)
    x2b = jnp.maximum(s1_ref[1] * dinv + b1_ref[0, hw:], 0.0)
    h2 = (jnp.dot(x2a, w2_ref[:hw], preferred_element_type=jnp.float32,
                  precision=lax.Precision.HIGHEST)
          + jnp.dot(x2b, w2_ref[hw:], preferred_element_type=jnp.float32,
                    precision=lax.Precision.HIGHEST))
    out_ref[...] = h2 * dinv


def _tc_post_body(s2_ref, g2_ref, dinv_ref, b2_ref, out_ref):
    s2 = s2_ref[0] + s2_ref[1] - g2_ref[...]
    out_ref[...] = jnp.maximum(s2 * dinv_ref[...] + b2_ref[0], 0.0)


_tc_pre = pl.pallas_call(
    _tc_pre_body,
    grid=(G,),
    in_specs=[
        pl.BlockSpec((R, D_IN), lambda r: (r, 0)),
        pl.BlockSpec((D_IN, D_HID), lambda r: (0, 0)),
        pl.BlockSpec((R, 1), lambda r: (r, 0)),
    ],
    out_specs=pl.BlockSpec((NSC, R, D_HID // 2), lambda r: (0, r, 0)),
    out_shape=jax.ShapeDtypeStruct((NSC, NPAD, D_HID // 2), jnp.float32),
)
_tc_mid = pl.pallas_call(
    _tc_mid_body,
    grid=(G,),
    in_specs=[
        pl.BlockSpec((NSC, R, D_HID // 2), lambda r: (0, r, 0)),
        pl.BlockSpec((R, 1), lambda r: (r, 0)),
        pl.BlockSpec((1, D_HID), lambda r: (0, 0)),
        pl.BlockSpec((D_HID, D_OUT), lambda r: (0, 0)),
    ],
    out_specs=pl.BlockSpec((R, D_OUT), lambda r: (r, 0)),
    out_shape=jax.ShapeDtypeStruct((NPAD, D_OUT), jnp.float32),
)
_tc_post = pl.pallas_call(
    _tc_post_body,
    grid=(G,),
    in_specs=[
        pl.BlockSpec((NSC, R, D_OUT), lambda r: (0, r, 0)),
        pl.BlockSpec((R, D_OUT), lambda r: (r, 0)),
        pl.BlockSpec((R, 1), lambda r: (r, 0)),
        pl.BlockSpec((1, D_OUT), lambda r: (0, 0)),
    ],
    out_specs=pl.BlockSpec((R, D_OUT), lambda r: (r, 0)),
    out_shape=jax.ShapeDtypeStruct((NPAD, D_OUT), jnp.float32),
)


def kernel(x, edge_index, W1, b1, W2, b2):
    ei = edge_index.astype(jnp.int32)
    src = ei[0]
    dst = ei[1]
    idx = jnp.concatenate([src, src + NPAD, dst])
    ones_c = jnp.ones((K, HIST_W), jnp.float32)
    zeros_c = jnp.zeros((NPAD, HIST_W), jnp.float32)

    hist = _sc_hist(idx, ones_c, zeros_c)
    deg = 1.0 + hist[0, :, 0] + hist[1, :, 0]
    dinv = lax.rsqrt(deg)[:, None]

    x_pad = jnp.zeros((NPAD, D_IN), x.dtype).at[:N].set(x)
    g1 = _tc_pre(x_pad, W1, dinv)
    s1 = _sc_edge_l1(g1.reshape(2 * NPAD, D_HID // 2), idx)
    g2 = _tc_mid(s1, dinv, b1.reshape(1, -1), W2)
    s2 = _sc_edge_l2(g2, idx)
    return _tc_post(s2, g2, dinv, b2.reshape(1, -1))[:N]
